# baseline (speedup 1.0000x reference)
import functools

import jax
import jax.numpy as jnp
from jax import lax
from jax.experimental import pallas as pl
from jax.experimental.pallas import tpu as pltpu
from jax.experimental.pallas import tpu_sc as plsc

_N = 10000
_E = 320000
_D = 128
_NC, _NS = 2, 16
_NW = _NC * _NS
_NCHUNK = _E // 128
_CPW = _NCHUNK // _NW
_XTRA = _NCHUNK - _CPW * _NW
_HP = 10240

@functools.cache
def _make_sc_hist():
    mesh = plsc.VectorSubcoreMesh(
        core_axis_name="c", subcore_axis_name="s", num_cores=_NC, num_subcores=_NS
    )
    return functools.partial(
        pl.kernel,
        out_type=jax.ShapeDtypeStruct((_NC, _HP), jnp.float32),
        mesh=mesh,
        scratch_types=[
            pltpu.VMEM((_CPW * 128,), jnp.int32),
            pltpu.VMEM((128,), jnp.int32),
            pltpu.VMEM((_CPW * 128,), jnp.float32),
            pltpu.VMEM((640,), jnp.float32),
            pltpu.VMEM_SHARED((_HP,), jnp.float32),
            pltpu.SemaphoreType.DMA,
        ],
    )(_sc_hist_body)


def _sc_hist_body(idx_hbm, out_hbm, idx_v, idxx_v, ones_v, zeros_v, hist_sh, sem1):
    cid = lax.axis_index("c")
    sid = lax.axis_index("s")
    wid = sid * _NC + cid
    base = (_CPW * wid + lax.min(wid, _XTRA)) * 128

    cp_idx = pltpu.async_copy(idx_hbm.at[0, pl.ds(base, _CPW * 128)], idx_v, sem1)

    def fill(j, carry):
        for k in range(8):
            ones_v[pl.ds(j * 128 + k * 16, 16)] = jnp.full((16,), 1.0, jnp.float32)
        return carry

    lax.fori_loop(0, _CPW, fill, 0)
    for k in range(40):
        zeros_v[pl.ds(k * 16, 16)] = jnp.zeros((16,), jnp.float32)

    pltpu.sync_copy(zeros_v, hist_sh.at[pl.ds(sid * 640, 640)])

    @pl.when(wid < _XTRA)
    def _():
        pltpu.sync_copy(idx_hbm.at[0, pl.ds(base + _CPW * 128, 128)], idxx_v)

    cp_idx.wait()
    plsc.subcore_barrier()

    pltpu.sync_copy(ones_v, hist_sh.at[idx_v], add=True)

    @pl.when(wid < _XTRA)
    def _():
        pltpu.sync_copy(ones_v.at[pl.ds(0, 128)], hist_sh.at[idxx_v], add=True)

    plsc.subcore_barrier()

    @pl.when(sid == 0)
    def _():
        pltpu.sync_copy(hist_sh, out_hbm.at[cid])


_BLK = 10240


def _tc_matmul_body(x_ref, w_ref, b_ref, y_ref):
    y_ref[...] = lax.dot_general(
        x_ref[...], w_ref[...],
        (((1,), (1,)), ((), ())),
        preferred_element_type=jnp.float32,
    ) + b_ref[...]


def _tc_matmul(x, w, b, interpret=False):
    return pl.pallas_call(
        _tc_matmul_body,
        grid=(pl.cdiv(_N, _BLK),),
        in_specs=[
            pl.BlockSpec((_BLK, _D), lambda i: (i, 0)),
            pl.BlockSpec((_D, _D), lambda i: (0, 0)),
            pl.BlockSpec((1, _D), lambda i: (0, 0)),
        ],
        out_specs=pl.BlockSpec((_BLK, _D), lambda i: (i, 0)),
        out_shape=jax.ShapeDtypeStruct((_N, _D), jnp.float32),
        interpret=interpret,
    )(x, w, b)


def _tc_scale_body(y_ref, p_ref, z_ref):
    deg = p_ref[0, :] + p_ref[1, :]
    z_ref[...] = y_ref[...] * deg[:, None]


def _tc_scale(y, partial, interpret=False):
    return pl.pallas_call(
        _tc_scale_body,
        grid=(pl.cdiv(_N, _BLK),),
        in_specs=[
            pl.BlockSpec((_BLK, _D), lambda i: (i, 0)),
            pl.BlockSpec((_NC, _BLK), lambda i: (0, i)),
        ],
        out_specs=pl.BlockSpec((_BLK, _D), lambda i: (i, 0)),
        out_shape=jax.ShapeDtypeStruct((_N, _D), jnp.float32),
        interpret=interpret,
    )(y, partial)


def _tc_fused_body(x_ref, w_ref, b_ref, p_ref, z_ref):
    y = lax.dot_general(
        x_ref[...], w_ref[...],
        (((1,), (1,)), ((), ())),
        preferred_element_type=jnp.float32,
    ) + b_ref[...]
    deg = p_ref[0, :] + p_ref[1, :]
    z_ref[...] = y * deg[:, None]


def _tc_fused(x, w, b, partial, interpret=False):
    return pl.pallas_call(
        _tc_fused_body,
        grid=(pl.cdiv(_N, _BLK),),
        in_specs=[
            pl.BlockSpec((_BLK, _D), lambda i: (i, 0)),
            pl.BlockSpec((_D, _D), lambda i: (0, 0)),
            pl.BlockSpec((1, _D), lambda i: (0, 0)),
            pl.BlockSpec((_NC, _BLK), lambda i: (0, i)),
        ],
        out_specs=pl.BlockSpec((_BLK, _D), lambda i: (i, 0)),
        out_shape=jax.ShapeDtypeStruct((_N, _D), jnp.float32),
        interpret=interpret,
    )(x, w, b, partial)


def kernel(node_features, edge_node_indices, edge_features, W_fn, b_fn, W_fe, b_fe, W_fa, b_fa):
    partial = _make_sc_hist()(edge_node_indices.astype(jnp.int32))
    return _tc_fused(node_features, W_fn, b_fn.reshape(1, _D), partial)


# speedup vs baseline: 1.0406x; 1.0406x over previous
import functools

import jax
import jax.numpy as jnp
from jax import lax
from jax.experimental import pallas as pl
from jax.experimental.pallas import tpu as pltpu
from jax.experimental.pallas import tpu_sc as plsc

_N = 10000
_E = 320000
_D = 128
_NC, _NS = 2, 16
_NW = _NC * _NS
_NCHUNK = _E // 128
_CPW = _NCHUNK // _NW
_XTRA = _NCHUNK - _CPW * _NW
_HP = 10240

@functools.cache
def _make_sc_hist():
    mesh = plsc.VectorSubcoreMesh(
        core_axis_name="c", subcore_axis_name="s", num_cores=_NC, num_subcores=_NS
    )
    return functools.partial(
        pl.kernel,
        out_type=jax.ShapeDtypeStruct((_NC, _HP), jnp.float32),
        mesh=mesh,
        scratch_types=[
            pltpu.VMEM((_CPW * 128,), jnp.int32),
            pltpu.VMEM((128,), jnp.int32),
            pltpu.VMEM((_CPW * 128,), jnp.float32),
            pltpu.VMEM((640,), jnp.float32),
            pltpu.VMEM_SHARED((_HP,), jnp.float32),
            pltpu.SemaphoreType.DMA,
        ],
    )(_sc_hist_body)


def _sc_hist_body(idx_hbm, out_hbm, idx_v, idxx_v, ones_v, zeros_v, hist_sh, sem1):
    cid = lax.axis_index("c")
    sid = lax.axis_index("s")
    wid = sid * _NC + cid
    base = (_CPW * wid + lax.min(wid, _XTRA)) * 128

    cp_idx = pltpu.async_copy(idx_hbm.at[0, pl.ds(base, _CPW * 128)], idx_v, sem1)

    def fill(j, carry):
        for k in range(8):
            ones_v[pl.ds(j * 128 + k * 16, 16)] = jnp.full((16,), 1.0, jnp.float32)
        return carry

    lax.fori_loop(0, _CPW, fill, 0)
    for k in range(40):
        zeros_v[pl.ds(k * 16, 16)] = jnp.zeros((16,), jnp.float32)

    pltpu.sync_copy(zeros_v, hist_sh.at[pl.ds(sid * 640, 640)])

    @pl.when(wid < _XTRA)
    def _():
        pltpu.sync_copy(idx_hbm.at[0, pl.ds(base + _CPW * 128, 128)], idxx_v)

    cp_idx.wait()
    plsc.subcore_barrier()

    pltpu.sync_copy(ones_v, hist_sh.at[idx_v], add=True)

    @pl.when(wid < _XTRA)
    def _():
        pltpu.sync_copy(ones_v.at[pl.ds(0, 128)], hist_sh.at[idxx_v], add=True)

    plsc.subcore_barrier()

    @pl.when(sid == 0)
    def _():
        pltpu.sync_copy(hist_sh, out_hbm.at[cid])


_BLK = 5120


def _tc_fused_body(x_ref, w_ref, b_ref, p_ref, z_ref):
    y = lax.dot_general(
        x_ref[...], w_ref[...],
        (((1,), (1,)), ((), ())),
        preferred_element_type=jnp.float32,
    ) + b_ref[...]
    deg = p_ref[0, :] + p_ref[1, :]
    z_ref[...] = y * deg[:, None]


def _tc_fused(x, w, b, partial, interpret=False):
    return pl.pallas_call(
        _tc_fused_body,
        grid=(pl.cdiv(_N, _BLK),),
        in_specs=[
            pl.BlockSpec((_BLK, _D), lambda i: (i, 0)),
            pl.BlockSpec((_D, _D), lambda i: (0, 0)),
            pl.BlockSpec((1, _D), lambda i: (0, 0)),
            pl.BlockSpec((_NC, _BLK), lambda i: (0, i)),
        ],
        out_specs=pl.BlockSpec((_BLK, _D), lambda i: (i, 0)),
        out_shape=jax.ShapeDtypeStruct((_N, _D), jnp.float32),
        interpret=interpret,
    )(x, w, b, partial)


def kernel(node_features, edge_node_indices, edge_features, W_fn, b_fn, W_fe, b_fe, W_fa, b_fa):
    partial = _make_sc_hist()(edge_node_indices.astype(jnp.int32))
    return _tc_fused(node_features, W_fn, b_fn.reshape(1, _D), partial)
